# baseline (speedup 1.0000x reference)
import jax
import jax.numpy as jnp
from jax import lax
from jax.experimental import pallas as pl
from jax.experimental.pallas import tpu as pltpu
from jax.experimental.pallas import tpu_sc as plsc

N = 10000
E = 320000
D = 128
C = 40
CP = 48

NC = 2
NS = 16
NW = NC * NS
CHUNK = 128
NCHUNK = 80
EW = NCHUNK * CHUNK
ER = NW * NCHUNK
NP = 10240
RPT = NP // NS
NBUF = 3
UNROLL = 5
RPW = NP // NW

_SC_MESH = plsc.VectorSubcoreMesh(core_axis_name="c", subcore_axis_name="s",
                                  num_cores=NC, num_subcores=NS)
_SC_PARAMS = pltpu.CompilerParams(use_tc_tiling_on_sc=False)


def _matmul_body(x_ref, wt_ref, o_ref):
    o_ref[...] = jnp.dot(x_ref[...], wt_ref[...],
                         preferred_element_type=jnp.float32)


def _tc_matmul(x, wt):
    bm = 1000
    return pl.pallas_call(
        _matmul_body,
        grid=(N // bm,),
        in_specs=[
            pl.BlockSpec((bm, D), lambda i: (i, 0)),
            pl.BlockSpec((D, CP), lambda i: (0, 0)),
        ],
        out_specs=pl.BlockSpec((bm, CP), lambda i: (i, 0)),
        out_shape=jax.ShapeDtypeStruct((N, CP), jnp.float32),
    )(x, wt)


def _sc_combine_body(p_hbm, b_hbm, out_hbm, a_v, b_v, bias_v):
    cid = lax.axis_index("c")
    sid = lax.axis_index("s")
    wid = sid * NC + cid
    base = wid * RPW
    pltpu.sync_copy(p_hbm.at[0, pl.ds(base, RPW)], a_v)
    pltpu.sync_copy(p_hbm.at[1, pl.ds(base, RPW)], b_v)
    pltpu.sync_copy(b_hbm, bias_v)
    bias = [bias_v[pl.ds(cc * 16, 16)] for cc in range(CP // 16)]

    def step(i, carry):
        r0 = i * 16
        for j in range(16):
            for cc in range(CP // 16):
                sl = pl.ds(cc * 16, 16)
                a_v[r0 + j, sl] = a_v[r0 + j, sl] + b_v[r0 + j, sl] + bias[cc]
        return carry

    lax.fori_loop(0, RPW // 16, step, 0)
    pltpu.sync_copy(a_v, out_hbm.at[pl.ds(base, RPW)])


_sc_combine = pl.kernel(
    _sc_combine_body,
    out_type=jax.ShapeDtypeStruct((NP, CP), jnp.float32),
    mesh=_SC_MESH,
    compiler_params=_SC_PARAMS,
    scratch_types=[
        pltpu.VMEM((RPW, CP), jnp.float32),
        pltpu.VMEM((RPW, CP), jnp.float32),
        pltpu.VMEM((CP,), jnp.float32),
    ],
)


def _sc_round_body(y_hbm, src_hbm, dst_hbm, w_hbm, out_hbm,
                   src_v, dst_v, w_v, rows_v, acc, ycache, semg, sems):
    cid = lax.axis_index("c")
    sid = lax.axis_index("s")
    wid = sid * NC + cid

    ebase = wid * NCHUNK
    pltpu.sync_copy(src_hbm.at[pl.ds(ebase, NCHUNK)], src_v)
    pltpu.sync_copy(dst_hbm.at[pl.ds(ebase, NCHUNK)], dst_v)
    pltpu.sync_copy(w_hbm.at[pl.ds(ebase, NCHUNK)], w_v)

    base = sid * RPT
    ny = y_hbm.shape[0]
    if ny >= NP:
        pltpu.sync_copy(y_hbm.at[pl.ds(base, RPT)],
                        ycache.at[pl.ds(base, RPT)])
    else:
        rem = ny - (NS - 1) * RPT

        @pl.when(sid < NS - 1)
        def _():
            pltpu.sync_copy(y_hbm.at[pl.ds(base, RPT)],
                            ycache.at[pl.ds(base, RPT)])

        @pl.when(sid == NS - 1)
        def _():
            pltpu.sync_copy(y_hbm.at[pl.ds((NS - 1) * RPT, rem)],
                            ycache.at[pl.ds((NS - 1) * RPT, rem)])

    for cc in range(CP // 16):
        for r in range(CHUNK):
            rows_v[0, r, pl.ds(cc * 16, 16)] = jnp.zeros((16,), jnp.float32)
    for i in range(RPT // CHUNK):
        pltpu.sync_copy(rows_v.at[0], acc.at[pl.ds(base + i * CHUNK, CHUNK)])
    plsc.subcore_barrier()

    def g_start(c, b):
        return pltpu.async_copy(ycache.at[src_v.at[c]], rows_v.at[b],
                                semg.at[b])

    def s_start(c, b):
        return pltpu.async_copy(rows_v.at[b], acc.at[dst_v.at[c]],
                                sems.at[b], add=True)

    def scale(c, b):
        for g in range(CHUNK // 16):
            wv = w_v[c, pl.ds(g * 16, 16)]
            for j in range(16):
                e = g * 16 + j
                wj = wv[j]
                for cc in range(CP // 16):
                    sl = pl.ds(cc * 16, 16)
                    rows_v[b, e, sl] = rows_v[b, e, sl] * wj

    def step(i, carry):
        c0 = i * UNROLL
        dg = [g_start(c0 + u, u % NBUF) for u in range(NBUF)]
        ds = [None] * NBUF
        for u in range(UNROLL):
            b = u % NBUF
            if u >= 1 and u - 1 + NBUF < UNROLL:
                pb = (u - 1) % NBUF
                ds[pb].wait()
                dg[pb] = g_start(c0 + u - 1 + NBUF, pb)
            dg[b].wait()
            scale(c0 + u, b)
            ds[b] = s_start(c0 + u, b)
        for b in range(min(NBUF, UNROLL)):
            ds[b].wait()
        return carry

    lax.fori_loop(0, NCHUNK // UNROLL, step, 0)
    plsc.subcore_barrier()

    pltpu.sync_copy(acc.at[pl.ds(base, RPT)], out_hbm.at[cid, pl.ds(base, RPT)])


_sc_round = pl.kernel(
    _sc_round_body,
    out_type=jax.ShapeDtypeStruct((NC, NP, CP), jnp.float32),
    mesh=_SC_MESH,
    compiler_params=_SC_PARAMS,
    scratch_types=[
        pltpu.VMEM((NCHUNK, CHUNK), jnp.int32),
        pltpu.VMEM((NCHUNK, CHUNK), jnp.int32),
        pltpu.VMEM((NCHUNK, CHUNK), jnp.float32),
        pltpu.VMEM((NBUF, CHUNK, CP), jnp.float32),
        pltpu.VMEM_SHARED((NP, CP), jnp.float32),
        pltpu.VMEM_SHARED((NP, CP), jnp.float32),
        pltpu.SemaphoreType.DMA((NBUF,)),
        pltpu.SemaphoreType.DMA((NBUF,)),
    ],
)


def kernel(x, edge_index, edge_weight, W, b):
    pad = ER * CHUNK - E
    pad_idx = jnp.arange(pad, dtype=jnp.int32) % N
    src = jnp.concatenate([edge_index[0], pad_idx]).reshape(ER, CHUNK)
    dst = jnp.concatenate([edge_index[1], pad_idx]).reshape(ER, CHUNK)
    w2 = jnp.concatenate(
        [edge_weight, jnp.zeros((pad,), jnp.float32)]).reshape(ER, CHUNK)
    wt = jnp.zeros((D, CP), jnp.float32).at[:, :C].set(W.T)
    bp = jnp.zeros((CP,), jnp.float32).at[:C].set(b)
    zb = jnp.zeros((CP,), jnp.float32)

    y = _tc_matmul(x, wt)
    p = _sc_round(y, src, dst, w2)
    r = _sc_combine(p, zb)
    q = _sc_round(r, src, dst, w2)
    o = _sc_combine(q, bp)
    return o[:N, :C]


# speedup vs baseline: 1.1335x; 1.1335x over previous
import jax
import jax.numpy as jnp
from jax import lax
from jax.experimental import pallas as pl
from jax.experimental.pallas import tpu as pltpu
from jax.experimental.pallas import tpu_sc as plsc

N = 10000
E = 320000
D = 128
C = 40
CP = 48

NC = 2
NS = 16
NW = NC * NS
CHUNK = 80
NCHUNK = 125
EW = NCHUNK * CHUNK
NP = 10240
RPT = NP // NS
NBUF = 3
UNROLL = 5
RPW = NP // NW

_SC_MESH = plsc.VectorSubcoreMesh(core_axis_name="c", subcore_axis_name="s",
                                  num_cores=NC, num_subcores=NS)
_SC_PARAMS = pltpu.CompilerParams(use_tc_tiling_on_sc=False)


def _matmul_body(x_ref, wt_ref, o_ref):
    o_ref[...] = jnp.dot(x_ref[...], wt_ref[...],
                         preferred_element_type=jnp.float32)


def _tc_matmul(x, wt):
    bm = 1000
    return pl.pallas_call(
        _matmul_body,
        grid=(N // bm,),
        in_specs=[
            pl.BlockSpec((bm, D), lambda i: (i, 0)),
            pl.BlockSpec((D, CP), lambda i: (0, 0)),
        ],
        out_specs=pl.BlockSpec((bm, CP), lambda i: (i, 0)),
        out_shape=jax.ShapeDtypeStruct((N, CP), jnp.float32),
    )(x, wt)


def _sc_combine_body(p_hbm, b_hbm, out_hbm, a_v, b_v, bias_v):
    cid = lax.axis_index("c")
    sid = lax.axis_index("s")
    wid = sid * NC + cid
    base = wid * RPW
    pltpu.sync_copy(p_hbm.at[0, pl.ds(base, RPW)], a_v)
    pltpu.sync_copy(p_hbm.at[1, pl.ds(base, RPW)], b_v)
    pltpu.sync_copy(b_hbm, bias_v)
    bias = [bias_v[pl.ds(cc * 16, 16)] for cc in range(CP // 16)]

    def step(i, carry):
        r0 = i * 16
        for j in range(16):
            for cc in range(CP // 16):
                sl = pl.ds(cc * 16, 16)
                a_v[r0 + j, sl] = a_v[r0 + j, sl] + b_v[r0 + j, sl] + bias[cc]
        return carry

    lax.fori_loop(0, RPW // 16, step, 0)
    pltpu.sync_copy(a_v, out_hbm.at[pl.ds(base, RPW)])


_sc_combine = pl.kernel(
    _sc_combine_body,
    out_type=jax.ShapeDtypeStruct((NP, CP), jnp.float32),
    mesh=_SC_MESH,
    compiler_params=_SC_PARAMS,
    scratch_types=[
        pltpu.VMEM((RPW, CP), jnp.float32),
        pltpu.VMEM((RPW, CP), jnp.float32),
        pltpu.VMEM((CP,), jnp.float32),
    ],
)


def _sc_round_body(y_hbm, src_hbm, dst_hbm, w_hbm, out_hbm,
                   src_v, dst_v, w_v, rows_v, acc, semg, sems):
    cid = lax.axis_index("c")
    sid = lax.axis_index("s")
    wid = sid * NC + cid

    pltpu.sync_copy(src_hbm.at[wid], src_v)
    pltpu.sync_copy(dst_hbm.at[wid], dst_v)
    pltpu.sync_copy(w_hbm.at[wid], w_v)

    base = sid * RPT
    for cc in range(CP // 16):
        for r in range(CHUNK):
            rows_v[0, r, pl.ds(cc * 16, 16)] = jnp.zeros((16,), jnp.float32)
    for i in range(RPT // CHUNK):
        pltpu.sync_copy(rows_v.at[0], acc.at[pl.ds(base + i * CHUNK, CHUNK)])
    plsc.subcore_barrier()

    def g_start(c, b):
        return pltpu.async_copy(y_hbm.at[src_v.at[c]], rows_v.at[b],
                                semg.at[b])

    def s_start(c, b):
        return pltpu.async_copy(rows_v.at[b], acc.at[dst_v.at[c]],
                                sems.at[b], add=True)

    def scale(c, b):
        for g in range(CHUNK // 16):
            wv = w_v[c, pl.ds(g * 16, 16)]
            for j in range(16):
                e = g * 16 + j
                wj = wv[j]
                for cc in range(CP // 16):
                    sl = pl.ds(cc * 16, 16)
                    rows_v[b, e, sl] = rows_v[b, e, sl] * wj

    def step(i, carry):
        c0 = i * UNROLL
        dg = [g_start(c0 + u, u % NBUF) for u in range(NBUF)]
        ds = [None] * NBUF
        for u in range(UNROLL):
            b = u % NBUF
            if u >= 1 and u - 1 + NBUF < UNROLL:
                pb = (u - 1) % NBUF
                ds[pb].wait()
                dg[pb] = g_start(c0 + u - 1 + NBUF, pb)
            dg[b].wait()
            scale(c0 + u, b)
            ds[b] = s_start(c0 + u, b)
        for b in range(min(NBUF, UNROLL)):
            ds[b].wait()
        return carry

    lax.fori_loop(0, NCHUNK // UNROLL, step, 0)
    plsc.subcore_barrier()

    pltpu.sync_copy(acc.at[pl.ds(base, RPT)], out_hbm.at[cid, pl.ds(base, RPT)])


_sc_round = pl.kernel(
    _sc_round_body,
    out_type=jax.ShapeDtypeStruct((NC, NP, CP), jnp.float32),
    mesh=_SC_MESH,
    compiler_params=_SC_PARAMS,
    scratch_types=[
        pltpu.VMEM((NCHUNK, CHUNK), jnp.int32),
        pltpu.VMEM((NCHUNK, CHUNK), jnp.int32),
        pltpu.VMEM((NCHUNK, CHUNK), jnp.float32),
        pltpu.VMEM((NBUF, CHUNK, CP), jnp.float32),
        pltpu.VMEM_SHARED((NP, CP), jnp.float32),
        pltpu.SemaphoreType.DMA((NBUF,)),
        pltpu.SemaphoreType.DMA((NBUF,)),
    ],
)


def kernel(x, edge_index, edge_weight, W, b):
    src = edge_index[0].reshape(NW, NCHUNK, CHUNK)
    dst = edge_index[1].reshape(NW, NCHUNK, CHUNK)
    w2 = edge_weight.reshape(NW, NCHUNK, CHUNK)
    wt = jnp.zeros((D, CP), jnp.float32).at[:, :C].set(W.T)
    bp = jnp.zeros((CP,), jnp.float32).at[:C].set(b)
    zb = jnp.zeros((CP,), jnp.float32)

    y = _tc_matmul(x, wt)
    p = _sc_round(y, src, dst, w2)
    r = _sc_combine(p, zb)
    q = _sc_round(r, src, dst, w2)
    o = _sc_combine(q, bp)
    return o[:N, :C]


# speedup vs baseline: 1.1953x; 1.0545x over previous
import jax
import jax.numpy as jnp
from jax import lax
from jax.experimental import pallas as pl
from jax.experimental.pallas import tpu as pltpu
from jax.experimental.pallas import tpu_sc as plsc

N = 10000
E = 320000
D = 128
C = 40
CP = 48

NC = 2
NS = 16
NW = NC * NS
CHUNK = 80
NCHUNK = 125
EW = NCHUNK * CHUNK
NP = 10240
RPT = NP // NS
NBUF = 4
UNROLL = 5
RPW = NP // NW

_SC_MESH = plsc.VectorSubcoreMesh(core_axis_name="c", subcore_axis_name="s",
                                  num_cores=NC, num_subcores=NS)
_SC_PARAMS = pltpu.CompilerParams(use_tc_tiling_on_sc=False)


def _matmul_body(x_ref, wt_ref, o_ref):
    o_ref[...] = jnp.dot(x_ref[...], wt_ref[...],
                         preferred_element_type=jnp.float32)


def _tc_matmul(x, wt):
    bm = 1000
    return pl.pallas_call(
        _matmul_body,
        grid=(N // bm,),
        in_specs=[
            pl.BlockSpec((bm, D), lambda i: (i, 0)),
            pl.BlockSpec((D, CP), lambda i: (0, 0)),
        ],
        out_specs=pl.BlockSpec((bm, CP), lambda i: (i, 0)),
        out_shape=jax.ShapeDtypeStruct((N, CP), jnp.float32),
    )(x, wt)


def _sc_combine_body(p_hbm, b_hbm, out_hbm, a_v, b_v, bias_v):
    cid = lax.axis_index("c")
    sid = lax.axis_index("s")
    wid = sid * NC + cid
    base = wid * RPW
    pltpu.sync_copy(p_hbm.at[0, pl.ds(base, RPW)], a_v)
    pltpu.sync_copy(p_hbm.at[1, pl.ds(base, RPW)], b_v)
    pltpu.sync_copy(b_hbm, bias_v)
    bias = [bias_v[pl.ds(cc * 16, 16)] for cc in range(CP // 16)]

    def step(i, carry):
        r0 = i * 16
        for j in range(16):
            for cc in range(CP // 16):
                sl = pl.ds(cc * 16, 16)
                a_v[r0 + j, sl] = a_v[r0 + j, sl] + b_v[r0 + j, sl] + bias[cc]
        return carry

    lax.fori_loop(0, RPW // 16, step, 0)
    pltpu.sync_copy(a_v, out_hbm.at[pl.ds(base, RPW)])


_sc_combine = pl.kernel(
    _sc_combine_body,
    out_type=jax.ShapeDtypeStruct((NP, CP), jnp.float32),
    mesh=_SC_MESH,
    compiler_params=_SC_PARAMS,
    scratch_types=[
        pltpu.VMEM((RPW, CP), jnp.float32),
        pltpu.VMEM((RPW, CP), jnp.float32),
        pltpu.VMEM((CP,), jnp.float32),
    ],
)


def _sc_round_body(y_hbm, src_hbm, dst_hbm, w_hbm, out_hbm,
                   src_v, dst_v, w_v, rows_v, acc, semg, sems):
    cid = lax.axis_index("c")
    sid = lax.axis_index("s")
    wid = sid * NC + cid

    pltpu.sync_copy(src_hbm.at[wid], src_v)
    pltpu.sync_copy(dst_hbm.at[wid], dst_v)
    pltpu.sync_copy(w_hbm.at[wid], w_v)

    base = sid * RPT
    for cc in range(CP // 16):
        for r in range(CHUNK):
            rows_v[0, r, pl.ds(cc * 16, 16)] = jnp.zeros((16,), jnp.float32)
    for i in range(RPT // CHUNK):
        pltpu.sync_copy(rows_v.at[0], acc.at[pl.ds(base + i * CHUNK, CHUNK)])
    plsc.subcore_barrier()

    def g_start(c, b):
        return pltpu.async_copy(y_hbm.at[src_v.at[c]], rows_v.at[b],
                                semg.at[b])

    def s_start(c, b):
        return pltpu.async_copy(rows_v.at[b], acc.at[dst_v.at[c]],
                                sems.at[b], add=True)

    def scale(c, b):
        for g in range(CHUNK // 16):
            wv = w_v[c, pl.ds(g * 16, 16)]
            for j in range(16):
                e = g * 16 + j
                wj = wv[j]
                for cc in range(CP // 16):
                    sl = pl.ds(cc * 16, 16)
                    rows_v[b, e, sl] = rows_v[b, e, sl] * wj

    def step(i, carry):
        c0 = i * UNROLL
        dg = [g_start(c0 + u, u % NBUF) for u in range(NBUF)]
        ds = [None] * NBUF
        for u in range(UNROLL):
            b = u % NBUF
            if u >= 1 and u - 1 + NBUF < UNROLL:
                pb = (u - 1) % NBUF
                ds[pb].wait()
                dg[pb] = g_start(c0 + u - 1 + NBUF, pb)
            dg[b].wait()
            scale(c0 + u, b)
            ds[b] = s_start(c0 + u, b)
        for b in range(min(NBUF, UNROLL)):
            ds[b].wait()
        return carry

    lax.fori_loop(0, NCHUNK // UNROLL, step, 0)
    plsc.subcore_barrier()

    pltpu.sync_copy(acc.at[pl.ds(base, RPT)], out_hbm.at[cid, pl.ds(base, RPT)])


_sc_round = pl.kernel(
    _sc_round_body,
    out_type=jax.ShapeDtypeStruct((NC, NP, CP), jnp.float32),
    mesh=_SC_MESH,
    compiler_params=_SC_PARAMS,
    scratch_types=[
        pltpu.VMEM((NCHUNK, CHUNK), jnp.int32),
        pltpu.VMEM((NCHUNK, CHUNK), jnp.int32),
        pltpu.VMEM((NCHUNK, CHUNK), jnp.float32),
        pltpu.VMEM((NBUF, CHUNK, CP), jnp.float32),
        pltpu.VMEM_SHARED((NP, CP), jnp.float32),
        pltpu.SemaphoreType.DMA((NBUF,)),
        pltpu.SemaphoreType.DMA((NBUF,)),
    ],
)


def kernel(x, edge_index, edge_weight, W, b):
    src = edge_index[0].reshape(NW, NCHUNK, CHUNK)
    dst = edge_index[1].reshape(NW, NCHUNK, CHUNK)
    w2 = edge_weight.reshape(NW, NCHUNK, CHUNK)
    wt = jnp.zeros((D, CP), jnp.float32).at[:, :C].set(W.T)
    bp = jnp.zeros((CP,), jnp.float32).at[:C].set(b)
    zb = jnp.zeros((CP,), jnp.float32)

    y = _tc_matmul(x, wt)
    p = _sc_round(y, src, dst, w2)
    r = _sc_combine(p, zb)
    q = _sc_round(r, src, dst, w2)
    o = _sc_combine(q, bp)
    return o[:N, :C]
